# baseline (speedup 1.0000x reference)
import dataclasses
import functools

import jax
import jax.numpy as jnp
from jax import lax
from jax.experimental import pallas as pl
from jax.experimental.pallas import tpu as pltpu
from jax.experimental.pallas import tpu_sc as plsc

NUM_EMBEDS = 1000000
EMB_DIM = 64
BATCH = 16384

NC = 2
NS = 16
NW = NC * NS
N_TC = (NUM_EMBEDS + 127) // 128
TC_BASE = N_TC // NW
TC_EXTRA = N_TC % NW
N_BLK = BATCH // 16

_mesh = plsc.VectorSubcoreMesh(core_axis_name="c", subcore_axis_name="s")

_cp = pltpu.CompilerParams()
if "needs_layout_passes" in pltpu.CompilerParams.__dataclass_fields__:
    _cp = dataclasses.replace(_cp, needs_layout_passes=False)


def _gather_body(tab, idx_h, out, idx_v, wk_x, wk_k, chunk, stage, sem_c, sem_o):
    wid = lax.axis_index("s") * NC + lax.axis_index("c")
    iota = jnp.arange(16, dtype=jnp.int32)
    d_iota = [iota + 16 * g for g in range(4)]

    pltpu.sync_copy(idx_h, idx_v)

    tc0 = wid * TC_BASE + jnp.minimum(wid, TC_EXTRA)
    n_w = TC_BASE + (wid < TC_EXTRA).astype(jnp.int32)
    lo_w = tc0 * 128
    hi_w = jnp.minimum((tc0 + n_w) * 128, NUM_EMBEDS)

    def p1(b, off):
        vx = idx_v[pl.ds(b * 16, 16)]
        mb = (vx >= lo_w) & (vx < hi_w)
        mi = mb.astype(jnp.int32)
        cs = jnp.cumsum(mi)
        pos = off + cs - 1
        plsc.store_scatter(wk_x, [pos], vx, mask=mb)
        plsc.store_scatter(wk_k, [pos], iota + b * 16, mask=mb)
        return off + cs[15]

    m = lax.fori_loop(0, N_BLK, p1, jnp.int32(0))
    n_scan = (m + 15) // 16

    pltpu.async_copy(tab.at[:, pl.ds(lo_w, 128)], chunk.at[0], sem_c)

    def p2(c, h):
        sel = lax.rem(c, 2)
        lo_c = (tc0 + c) * 128
        hi_c = jnp.minimum(lo_c + 128, NUM_EMBEDS)

        @pl.when(c + 1 < n_w)
        def _():
            pltpu.async_copy(
                tab.at[:, pl.ds(lo_c + 128, 128)],
                chunk.at[lax.rem(c + 1, 2)],
                sem_c,
            )

        pltpu.make_async_copy(tab.at[:, pl.ds(0, 128)], chunk.at[0], sem_c).wait()

        def blk(b, h):
            base = b * 16
            vx = wk_x[pl.ds(base, 16)]
            mb = (vx >= lo_c) & (vx < hi_c) & ((base + iota) < m)
            mi = mb.astype(jnp.int32)
            cs = jnp.cumsum(mi)

            @pl.when(cs[15] > 0)
            def _():
                vk = wk_k[pl.ds(base, 16)]
                for u in range(16):
                    h_u = h + cs[u] - mi[u]

                    @pl.when(mi[u] != 0)
                    def _():
                        @pl.when(h_u >= 4)
                        def _():
                            pltpu.make_async_copy(
                                stage.at[pl.ds(0, 64)],
                                out.at[pl.ds(0, 64)],
                                sem_o,
                            ).wait()

                        col = jnp.full((16,), vx[u] - lo_c, jnp.int32)
                        slot = lax.rem(h_u, 4) * 64
                        for g in range(4):
                            vals = plsc.load_gather(
                                chunk,
                                [jnp.full((16,), sel, jnp.int32), d_iota[g], col],
                            )
                            stage[pl.ds(slot + g * 16, 16)] = vals
                        pltpu.async_copy(
                            stage.at[pl.ds(slot, 64)],
                            out.at[pl.ds(vk[u] * 64, 64)],
                            sem_o,
                        )

            return h + cs[15]

        return lax.fori_loop(0, n_scan, blk, h)

    h_tot = lax.fori_loop(0, n_w, p2, jnp.int32(0))

    def drain(_, carry):
        pltpu.make_async_copy(
            stage.at[pl.ds(0, 64)], out.at[pl.ds(0, 64)], sem_o
        ).wait()
        return carry

    lax.fori_loop(0, jnp.minimum(h_tot, 4), drain, jnp.int32(0))


@jax.jit
def kernel(x, table):
    tableT = table.T

    run = functools.partial(
        pl.kernel,
        mesh=_mesh,
        out_type=jax.ShapeDtypeStruct((BATCH * EMB_DIM,), jnp.float32),
        scratch_types=[
            pltpu.VMEM((BATCH,), jnp.int32),
            pltpu.VMEM((BATCH,), jnp.int32),
            pltpu.VMEM((BATCH,), jnp.int32),
            pltpu.VMEM((2, EMB_DIM, 128), jnp.float32),
            pltpu.VMEM((256,), jnp.float32),
            pltpu.SemaphoreType.DMA,
            pltpu.SemaphoreType.DMA,
        ],
        compiler_params=_cp,
    )(_gather_body)

    flat = run(tableT, x.astype(jnp.int32))
    return flat.reshape(BATCH, EMB_DIM)


# speedup vs baseline: 1.2624x; 1.2624x over previous
import dataclasses
import functools

import jax
import jax.numpy as jnp
from jax import lax
from jax.experimental import pallas as pl
from jax.experimental.pallas import tpu as pltpu
from jax.experimental.pallas import tpu_sc as plsc

NUM_EMBEDS = 1000000
EMB_DIM = 64
BATCH = 16384

NC = 2
NS = 16
NW = NC * NS
N_TC = (NUM_EMBEDS + 127) // 128
TC_BASE = N_TC // NW
TC_EXTRA = N_TC % NW
N_BLK = BATCH // 16
CHUNK = 256
NBUF = 4
WB_MAX = N_TC * 128 - CHUNK

_mesh = plsc.VectorSubcoreMesh(core_axis_name="c", subcore_axis_name="s")

_cp = pltpu.CompilerParams()
if "needs_layout_passes" in pltpu.CompilerParams.__dataclass_fields__:
    _cp = dataclasses.replace(_cp, needs_layout_passes=False)


def _gather_body(tab, idx_h, out, idx_v, wk_x, wk_k, chunk, stage, sem_c, sem_o):
    wid = lax.axis_index("s") * NC + lax.axis_index("c")
    iota = jnp.arange(16, dtype=jnp.int32)
    d_iota = [iota + 16 * g for g in range(4)]

    pltpu.sync_copy(idx_h, idx_v)

    tc0 = wid * TC_BASE + jnp.minimum(wid, TC_EXTRA)
    n_w = TC_BASE + (wid < TC_EXTRA).astype(jnp.int32)
    lo_w = tc0 * 128
    hi_w = jnp.minimum((tc0 + n_w) * 128, NUM_EMBEDS)
    n_chunk = (n_w * 128 + CHUNK - 1) // CHUNK

    def window(c):
        lo_c = lo_w + c * CHUNK
        hi_c = jnp.minimum(lo_c + CHUNK, hi_w)
        wb = jnp.minimum(lo_c, WB_MAX)
        return lo_c, hi_c, wb

    def p1(b, off):
        vx = idx_v[pl.ds(b * 16, 16)]
        mb = (vx >= lo_w) & (vx < hi_w)
        mi = mb.astype(jnp.int32)
        cs = jnp.cumsum(mi)
        pos = off + cs - 1
        plsc.store_scatter(wk_x, [pos], vx, mask=mb)
        plsc.store_scatter(wk_k, [pos], iota + b * 16, mask=mb)
        return off + cs[15]

    m = lax.fori_loop(0, N_BLK, p1, jnp.int32(0))
    n_scan = (m + 15) // 16

    def issue(c):
        _, _, wb = window(c)
        pltpu.async_copy(
            tab.at[:, pl.ds(wb, CHUNK)], chunk.at[lax.rem(c, NBUF)], sem_c
        )

    for c0 in range(NBUF - 1):
        @pl.when(c0 < n_chunk)
        def _():
            issue(jnp.int32(c0))

    def p2(c, h):
        sel = lax.rem(c, NBUF)
        lo_c, hi_c, wb = window(c)

        @pl.when(c + (NBUF - 1) < n_chunk)
        def _():
            issue(c + (NBUF - 1))

        pltpu.make_async_copy(
            tab.at[:, pl.ds(0, CHUNK)], chunk.at[0], sem_c
        ).wait()

        def blk(b, h):
            base = b * 16
            vx = wk_x[pl.ds(base, 16)]
            mb = (vx >= lo_c) & (vx < hi_c) & ((base + iota) < m)
            mi = mb.astype(jnp.int32)
            cs = jnp.cumsum(mi)

            @pl.when(cs[15] > 0)
            def _():
                vk = wk_k[pl.ds(base, 16)]
                for u in range(16):
                    h_u = h + cs[u] - mi[u]

                    @pl.when(mi[u] != 0)
                    def _():
                        @pl.when(h_u >= 4)
                        def _():
                            pltpu.make_async_copy(
                                stage.at[pl.ds(0, 64)],
                                out.at[pl.ds(0, 64)],
                                sem_o,
                            ).wait()

                        col = jnp.full((16,), vx[u] - wb, jnp.int32)
                        slot = lax.rem(h_u, 4) * 64
                        for g in range(4):
                            vals = plsc.load_gather(
                                chunk,
                                [jnp.full((16,), sel, jnp.int32), d_iota[g], col],
                            )
                            stage[pl.ds(slot + g * 16, 16)] = vals
                        pltpu.async_copy(
                            stage.at[pl.ds(slot, 64)],
                            out.at[pl.ds(vk[u] * 64, 64)],
                            sem_o,
                        )

            return h + cs[15]

        return lax.fori_loop(0, n_scan, blk, h)

    h_tot = lax.fori_loop(0, n_chunk, p2, jnp.int32(0))

    def drain(_, carry):
        pltpu.make_async_copy(
            stage.at[pl.ds(0, 64)], out.at[pl.ds(0, 64)], sem_o
        ).wait()
        return carry

    lax.fori_loop(0, jnp.minimum(h_tot, 4), drain, jnp.int32(0))


@jax.jit
def kernel(x, table):
    tableT = table.T

    run = functools.partial(
        pl.kernel,
        mesh=_mesh,
        out_type=jax.ShapeDtypeStruct((BATCH * EMB_DIM,), jnp.float32),
        scratch_types=[
            pltpu.VMEM((BATCH,), jnp.int32),
            pltpu.VMEM((BATCH,), jnp.int32),
            pltpu.VMEM((BATCH,), jnp.int32),
            pltpu.VMEM((NBUF, EMB_DIM, CHUNK), jnp.float32),
            pltpu.VMEM((256,), jnp.float32),
            pltpu.SemaphoreType.DMA,
            pltpu.SemaphoreType.DMA,
        ],
        compiler_params=_cp,
    )(_gather_body)

    flat = run(tableT, x.astype(jnp.int32))
    return flat.reshape(BATCH, EMB_DIM)


# speedup vs baseline: 2.2108x; 1.7513x over previous
import dataclasses
import functools

import jax
import jax.numpy as jnp
from jax import lax
from jax.experimental import pallas as pl
from jax.experimental.pallas import tpu as pltpu
from jax.experimental.pallas import tpu_sc as plsc

NUM_EMBEDS = 1000000
EMB_DIM = 64
BATCH = 16384

NC = 2
NS = 16
NW = NC * NS
N_TC = (NUM_EMBEDS + 127) // 128
TC_BASE = N_TC // NW
TC_EXTRA = N_TC % NW
N_BLK = BATCH // 16
CHUNK = 256
CHUNK_SHIFT = 8
NBUF = 4
N_CHUNK_MAX = 128
WB_MAX = N_TC * 128 - CHUNK

_mesh = plsc.VectorSubcoreMesh(core_axis_name="c", subcore_axis_name="s")

_cp = pltpu.CompilerParams()
if "needs_layout_passes" in pltpu.CompilerParams.__dataclass_fields__:
    _cp = dataclasses.replace(_cp, needs_layout_passes=False)


def _gather_body(tab, idx_h, out, idx_v, wk_x, wk_k, cnt, cur, chunk, stage,
                 sem_c, sem_o):
    wid = lax.axis_index("s") * NC + lax.axis_index("c")
    iota = jnp.arange(16, dtype=jnp.int32)
    zeros16 = jnp.zeros((16,), jnp.int32)
    d_iota = [iota + 16 * g for g in range(4)]

    pltpu.sync_copy(idx_h, idx_v)

    tc0 = wid * TC_BASE + jnp.minimum(wid, TC_EXTRA)
    n_w = TC_BASE + (wid < TC_EXTRA).astype(jnp.int32)
    lo_w = tc0 * 128
    hi_w = jnp.minimum((tc0 + n_w) * 128, NUM_EMBEDS)
    n_chunk = (n_w * 128 + CHUNK - 1) // CHUNK

    for i in range(N_CHUNK_MAX // 16):
        cnt[pl.ds(i * 16, 16)] = zeros16

    def p1a(b, carry):
        vx = idx_v[pl.ds(b * 16, 16)]
        mb = (vx >= lo_w) & (vx < hi_w)
        cvec = lax.shift_right_logical(vx - lo_w, CHUNK_SHIFT)
        plsc.addupdate_scatter(cnt, [cvec], mb.astype(jnp.int32), mask=mb)
        return carry

    lax.fori_loop(0, N_BLK, p1a, jnp.int32(0))

    def prefix(i, acc):
        v = cnt[pl.ds(i * 16, 16)]
        inc = jnp.cumsum(v)
        cur[pl.ds(i * 16, 16)] = acc + inc - v
        return acc + inc[15]

    m = lax.fori_loop(0, N_CHUNK_MAX // 16, prefix, jnp.int32(0))

    def p1c(b, carry):
        vx = idx_v[pl.ds(b * 16, 16)]
        mb = (vx >= lo_w) & (vx < hi_w)
        mi = mb.astype(jnp.int32)
        cs = jnp.cumsum(mi)

        @pl.when(cs[15] > 0)
        def _():
            for u in range(16):
                @pl.when(mi[u] != 0)
                def _():
                    xv = vx[u]
                    cfull = jnp.full((16,), lax.shift_right_logical(
                        xv - lo_w, CHUNK_SHIFT), jnp.int32)
                    pvec = plsc.load_gather(cur, [cfull])
                    plsc.store_scatter(wk_x, [pvec], jnp.full((16,), xv, jnp.int32))
                    plsc.store_scatter(
                        wk_k, [pvec], jnp.full((16,), b * 16 + u, jnp.int32))
                    plsc.store_scatter(cur, [cfull], pvec + 1)

        return carry

    lax.fori_loop(0, N_BLK, p1c, jnp.int32(0))

    def window(c):
        lo_c = lo_w + c * CHUNK
        wb = jnp.minimum(lo_c, WB_MAX)
        return lo_c, wb

    def issue(c):
        _, wb = window(c)
        pltpu.async_copy(
            tab.at[:, pl.ds(wb, CHUNK)], chunk.at[lax.rem(c, NBUF)], sem_c
        )

    for c0 in range(NBUF - 1):
        @pl.when(c0 < n_chunk)
        def _():
            issue(jnp.int32(c0))

    def p2(c, h):
        sel = lax.rem(c, NBUF)
        _, wb = window(c)

        @pl.when(c + (NBUF - 1) < n_chunk)
        def _():
            issue(c + (NBUF - 1))

        pltpu.make_async_copy(
            tab.at[:, pl.ds(0, CHUNK)], chunk.at[0], sem_c
        ).wait()

        evec = plsc.load_gather(cur, [jnp.full((16,), c, jnp.int32)])
        nvec = plsc.load_gather(cnt, [jnp.full((16,), c, jnp.int32)])
        e = evec[0]
        s = e - nvec[0]
        p0 = lax.bitwise_and(s, jnp.int32(-16))
        nb = lax.shift_right_logical(e - p0 + 15, 4)

        def blk(b16, h):
            base = p0 + b16 * 16
            lane = base + iota
            vx = wk_x[pl.ds(base, 16)]
            mb = (lane >= s) & (lane < e)
            mi = mb.astype(jnp.int32)
            cs = jnp.cumsum(mi)

            @pl.when(cs[15] > 0)
            def _():
                vk = wk_k[pl.ds(base, 16)]
                for u in range(16):
                    h_u = h + cs[u] - mi[u]

                    @pl.when(mi[u] != 0)
                    def _():
                        @pl.when(h_u >= 4)
                        def _():
                            pltpu.make_async_copy(
                                stage.at[pl.ds(0, 64)],
                                out.at[pl.ds(0, 64)],
                                sem_o,
                            ).wait()

                        col = jnp.full((16,), vx[u] - wb, jnp.int32)
                        slot = lax.rem(h_u, 4) * 64
                        for g in range(4):
                            vals = plsc.load_gather(
                                chunk,
                                [jnp.full((16,), sel, jnp.int32), d_iota[g], col],
                            )
                            stage[pl.ds(slot + g * 16, 16)] = vals
                        pltpu.async_copy(
                            stage.at[pl.ds(slot, 64)],
                            out.at[pl.ds(vk[u] * 64, 64)],
                            sem_o,
                        )

            return h + cs[15]

        return lax.fori_loop(0, nb, blk, h)

    h_tot = lax.fori_loop(0, n_chunk, p2, jnp.int32(0))

    def drain(_, carry):
        pltpu.make_async_copy(
            stage.at[pl.ds(0, 64)], out.at[pl.ds(0, 64)], sem_o
        ).wait()
        return carry

    lax.fori_loop(0, jnp.minimum(h_tot, 4), drain, jnp.int32(0))


@jax.jit
def kernel(x, table):
    tableT = table.T

    run = functools.partial(
        pl.kernel,
        mesh=_mesh,
        out_type=jax.ShapeDtypeStruct((BATCH * EMB_DIM,), jnp.float32),
        scratch_types=[
            pltpu.VMEM((BATCH,), jnp.int32),
            pltpu.VMEM((BATCH + 16,), jnp.int32),
            pltpu.VMEM((BATCH + 16,), jnp.int32),
            pltpu.VMEM((N_CHUNK_MAX,), jnp.int32),
            pltpu.VMEM((N_CHUNK_MAX,), jnp.int32),
            pltpu.VMEM((NBUF, EMB_DIM, CHUNK), jnp.float32),
            pltpu.VMEM((256,), jnp.float32),
            pltpu.SemaphoreType.DMA,
            pltpu.SemaphoreType.DMA,
        ],
        compiler_params=_cp,
    )(_gather_body)

    flat = run(tableT, x.astype(jnp.int32))
    return flat.reshape(BATCH, EMB_DIM)


# speedup vs baseline: 2.2382x; 1.0124x over previous
import dataclasses
import functools

import jax
import jax.numpy as jnp
from jax import lax
from jax.experimental import pallas as pl
from jax.experimental.pallas import tpu as pltpu
from jax.experimental.pallas import tpu_sc as plsc

NUM_EMBEDS = 1000000
EMB_DIM = 64
BATCH = 16384

NC = 2
NS = 16
NW = NC * NS
N_TC = (NUM_EMBEDS + 127) // 128
TC_BASE = N_TC // NW
TC_EXTRA = N_TC % NW
N_BLK = BATCH // 16
CHUNK = 256
CHUNK_SHIFT = 8
NBUF = 5
N_CHUNK_MAX = 128
WB_MAX = N_TC * 128 - CHUNK

_mesh = plsc.VectorSubcoreMesh(core_axis_name="c", subcore_axis_name="s")

_cp = pltpu.CompilerParams()
if "needs_layout_passes" in pltpu.CompilerParams.__dataclass_fields__:
    _cp = dataclasses.replace(_cp, needs_layout_passes=False)


def _gather_body(tab, idx_h, out, idx_v, wk, cnt, cur, chunk, stage,
                 sem_c, sem_o):
    wid = lax.axis_index("s") * NC + lax.axis_index("c")
    iota = jnp.arange(16, dtype=jnp.int32)
    zeros16 = jnp.zeros((16,), jnp.int32)
    d_iota = [iota + 16 * g for g in range(4)]

    tc0 = wid * TC_BASE + jnp.minimum(wid, TC_EXTRA)
    n_w = TC_BASE + (wid < TC_EXTRA).astype(jnp.int32)
    lo_w = tc0 * 128
    hi_w = jnp.minimum((tc0 + n_w) * 128, NUM_EMBEDS)
    n_chunk = (n_w * 128 + CHUNK - 1) // CHUNK

    def window(c):
        lo_c = lo_w + c * CHUNK
        wb = jnp.minimum(lo_c, WB_MAX)
        return lo_c, wb

    def issue(c):
        _, wb = window(c)
        pltpu.async_copy(
            tab.at[:, pl.ds(wb, CHUNK)], chunk.at[lax.rem(c, NBUF)], sem_c
        )

    for c0 in range(NBUF - 1):
        @pl.when(c0 < n_chunk)
        def _():
            issue(jnp.int32(c0))

    pltpu.sync_copy(idx_h, idx_v)

    for i in range(N_CHUNK_MAX // 16):
        cnt[pl.ds(i * 16, 16)] = zeros16

    def p1a(b, carry):
        for s in range(4):
            vx = idx_v[pl.ds((b * 4 + s) * 16, 16)]
            mb = (vx >= lo_w) & (vx < hi_w)
            cvec = lax.shift_right_logical(vx - lo_w, CHUNK_SHIFT)
            plsc.addupdate_scatter(cnt, [cvec], mb.astype(jnp.int32), mask=mb)
        return carry

    lax.fori_loop(0, N_BLK // 4, p1a, jnp.int32(0))

    def prefix(i, acc):
        v = cnt[pl.ds(i * 16, 16)]
        inc = jnp.cumsum(v)
        cur[pl.ds(i * 16, 16)] = acc + inc - v
        return acc + inc[15]

    lax.fori_loop(0, N_CHUNK_MAX // 16, prefix, jnp.int32(0))

    def p1c(b, carry):
        for s in range(2):
            bb = b * 2 + s
            vx = idx_v[pl.ds(bb * 16, 16)]
            mb = (vx >= lo_w) & (vx < hi_w)
            mi = mb.astype(jnp.int32)
            npc = plsc.all_reduce_population_count(mb)

            @pl.when(npc[0] > 0)
            def _():
                for u in range(16):
                    @pl.when(mi[u] != 0)
                    def _():
                        xv = vx[u] - lo_w
                        cfull = jnp.full(
                            (16,),
                            lax.shift_right_logical(xv, CHUNK_SHIFT),
                            jnp.int32,
                        )
                        pvec = plsc.load_gather(cur, [cfull])
                        packed = ((bb * 16 + u) << 8) | lax.bitwise_and(
                            xv, jnp.int32(CHUNK - 1))
                        plsc.store_scatter(
                            wk, [pvec], jnp.full((16,), packed, jnp.int32))
                        plsc.store_scatter(cur, [cfull], pvec + 1)
        return carry

    lax.fori_loop(0, N_BLK // 2, p1c, jnp.int32(0))

    def p2(c, h):
        sel = lax.rem(c, NBUF)
        lo_c, wb = window(c)
        coladj = lo_c - wb

        @pl.when(c + (NBUF - 1) < n_chunk)
        def _():
            issue(c + (NBUF - 1))

        pltpu.make_async_copy(
            tab.at[:, pl.ds(0, CHUNK)], chunk.at[0], sem_c
        ).wait()

        cfull = jnp.full((16,), c, jnp.int32)
        e = plsc.load_gather(cur, [cfull])[0]
        s = e - plsc.load_gather(cnt, [cfull])[0]
        p0 = lax.bitwise_and(s, jnp.int32(-16))
        nb = lax.shift_right_logical(e - p0 + 15, 4)

        def blk(b16, h):
            base = p0 + b16 * 16
            lane = base + iota
            vw = wk[pl.ds(base, 16)]
            mb = (lane >= s) & (lane < e)
            mi = mb.astype(jnp.int32)
            cs = jnp.cumsum(mi)

            @pl.when(cs[15] > 0)
            def _():
                for u in range(16):
                    h_u = h + cs[u] - mi[u]

                    @pl.when(mi[u] != 0)
                    def _():
                        @pl.when(h_u >= 4)
                        def _():
                            pltpu.make_async_copy(
                                stage.at[pl.ds(0, 64)],
                                out.at[pl.ds(0, 64)],
                                sem_o,
                            ).wait()

                        w = vw[u]
                        col = jnp.full(
                            (16,),
                            lax.bitwise_and(w, jnp.int32(CHUNK - 1)) + coladj,
                            jnp.int32,
                        )
                        slot = lax.rem(h_u, 4) * 64
                        for g in range(4):
                            vals = plsc.load_gather(
                                chunk,
                                [jnp.full((16,), sel, jnp.int32), d_iota[g], col],
                            )
                            stage[pl.ds(slot + g * 16, 16)] = vals
                        k = lax.shift_right_logical(w, 8)
                        pltpu.async_copy(
                            stage.at[pl.ds(slot, 64)],
                            out.at[pl.ds(k * 64, 64)],
                            sem_o,
                        )

            return h + cs[15]

        return lax.fori_loop(0, nb, blk, h)

    h_tot = lax.fori_loop(0, n_chunk, p2, jnp.int32(0))

    def drain(_, carry):
        pltpu.make_async_copy(
            stage.at[pl.ds(0, 64)], out.at[pl.ds(0, 64)], sem_o
        ).wait()
        return carry

    lax.fori_loop(0, jnp.minimum(h_tot, 4), drain, jnp.int32(0))


@jax.jit
def kernel(x, table):
    tableT = table.T

    run = functools.partial(
        pl.kernel,
        mesh=_mesh,
        out_type=jax.ShapeDtypeStruct((BATCH * EMB_DIM,), jnp.float32),
        scratch_types=[
            pltpu.VMEM((BATCH,), jnp.int32),
            pltpu.VMEM((BATCH + 16,), jnp.int32),
            pltpu.VMEM((N_CHUNK_MAX,), jnp.int32),
            pltpu.VMEM((N_CHUNK_MAX,), jnp.int32),
            pltpu.VMEM((NBUF, EMB_DIM, CHUNK), jnp.float32),
            pltpu.VMEM((256,), jnp.float32),
            pltpu.SemaphoreType.DMA,
            pltpu.SemaphoreType.DMA,
        ],
        compiler_params=_cp,
    )(_gather_body)

    flat = run(tableT, x.astype(jnp.int32))
    return flat.reshape(BATCH, EMB_DIM)
